# Initial kernel scaffold; baseline (speedup 1.0000x reference)
import jax
import jax.numpy as jnp
from jax.experimental import pallas as pl


def kernel(feat_t0, feat_t1, ground_truth):
    raise NotImplementedError("write your pallas kernel here")


# speedup vs baseline: 2.0375x; 2.0375x over previous
import jax
import jax.numpy as jnp
from jax.experimental import pallas as pl
from jax.experimental.pallas import tpu as pltpu

_BINS = 100
_CHUNK = 8192


def _hist_loss_kernel(f0_ref, f1_ref, gt_ref, out_ref, hp_ref, ha_ref, sz_ref):
    i = pl.program_id(0)
    nsteps = pl.num_programs(0)

    @pl.when(i == 0)
    def _init():
        hp_ref[...] = jnp.zeros_like(hp_ref)
        ha_ref[...] = jnp.zeros_like(ha_ref)
        sz_ref[...] = jnp.zeros_like(sz_ref)

    d = f0_ref[...] - f1_ref[...] + 1e-6
    dist = jnp.sqrt(jnp.sum(d * d, axis=0, keepdims=True))
    gt = gt_ref[...]
    pos = (gt == 0).astype(jnp.float32)
    in_range = (dist <= 1.0).astype(jnp.float32)
    idx = jnp.clip(jnp.floor(dist * float(_BINS)).astype(jnp.int32), 0, _BINS - 1)
    bins = jax.lax.broadcasted_iota(jnp.int32, (_BINS, 1), 0)
    onehot = (idx == bins).astype(jnp.float32)
    hp_ref[...] += jnp.sum(onehot * (pos * in_range), axis=1, keepdims=True)
    ha_ref[...] += jnp.sum(onehot * in_range, axis=1, keepdims=True)
    sz_ref[...] += jnp.sum(pos).reshape(1, 1)

    @pl.when(i == nsteps - 1)
    def _finalize():
        npix = nsteps * _CHUNK
        pos_size = sz_ref[0, 0]
        neg_size = float(npix) - pos_size
        hp = hp_ref[...] / pos_size
        hn = (ha_ref[...] - hp_ref[...]) / neg_size
        pointwise = jnp.where(hn > 0, hn * (jnp.log(hn) - hp), 0.0)
        out_ref[...] = (jnp.sum(pointwise) / float(_BINS) + 1.0).reshape(1, 1)


@jax.jit
def kernel(feat_t0, feat_t1, ground_truth):
    n, c, h, w = feat_t0.shape
    npix = h * w
    f0 = feat_t0.reshape(c, npix)
    f1 = feat_t1.reshape(c, npix)
    gt = ground_truth.reshape(1, npix)
    grid = npix // _CHUNK
    out = pl.pallas_call(
        _hist_loss_kernel,
        grid=(grid,),
        in_specs=[
            pl.BlockSpec((c, _CHUNK), lambda i: (0, i)),
            pl.BlockSpec((c, _CHUNK), lambda i: (0, i)),
            pl.BlockSpec((1, _CHUNK), lambda i: (0, i)),
        ],
        out_specs=pl.BlockSpec((1, 1), lambda i: (0, 0)),
        out_shape=jax.ShapeDtypeStruct((1, 1), jnp.float32),
        scratch_shapes=[
            pltpu.VMEM((_BINS, 1), jnp.float32),
            pltpu.VMEM((_BINS, 1), jnp.float32),
            pltpu.VMEM((1, 1), jnp.float32),
        ],
        compiler_params=pltpu.CompilerParams(
            dimension_semantics=("arbitrary",),
        ),
    )(f0, f1, gt)
    return out[0, 0]


# speedup vs baseline: 2.1043x; 1.0328x over previous
import jax
import jax.numpy as jnp
from jax.experimental import pallas as pl
from jax.experimental.pallas import tpu as pltpu

_BINS = 100
_CHUNK = 8192


def _hist_loss_kernel(f0_ref, f1_ref, gt_ref, out_ref, hp_ref, ha_ref, sz_ref):
    i = pl.program_id(0)
    nsteps = pl.num_programs(0)

    @pl.when(i == 0)
    def _init():
        hp_ref[...] = jnp.zeros_like(hp_ref)
        ha_ref[...] = jnp.zeros_like(ha_ref)
        sz_ref[...] = jnp.zeros_like(sz_ref)

    t = f0_ref[...] - f1_ref[...]
    z = t * t
    c = t.shape[0]
    ones_row = jnp.ones((1, c), dtype=jnp.float32)
    sz = jnp.dot(ones_row, z, preferred_element_type=jnp.float32)
    st = jnp.dot(ones_row, t, preferred_element_type=jnp.float32)
    dist2 = sz + 2e-6 * st + float(c) * 1e-12
    dist = jnp.sqrt(dist2)
    gt = gt_ref[...]
    pos = (gt == 0).astype(jnp.float32)
    in_range = (dist <= 1.0).astype(jnp.float32)
    idx = jnp.clip(jnp.floor(dist * float(_BINS)).astype(jnp.int32), 0, _BINS - 1)
    bins = jax.lax.broadcasted_iota(jnp.int32, (_BINS, 1), 0)
    onehot = (idx == bins).astype(jnp.float32)
    hp_ref[...] += jnp.sum(onehot * (pos * in_range), axis=1, keepdims=True)
    ha_ref[...] += jnp.sum(onehot * in_range, axis=1, keepdims=True)
    sz_ref[...] += jnp.sum(pos).reshape(1, 1)

    @pl.when(i == nsteps - 1)
    def _finalize():
        npix = nsteps * _CHUNK
        pos_size = sz_ref[0, 0]
        neg_size = float(npix) - pos_size
        hp = hp_ref[...] / pos_size
        hn = (ha_ref[...] - hp_ref[...]) / neg_size
        pointwise = jnp.where(hn > 0, hn * (jnp.log(hn) - hp), 0.0)
        out_ref[...] = (jnp.sum(pointwise) / float(_BINS) + 1.0).reshape(1, 1)


@jax.jit
def kernel(feat_t0, feat_t1, ground_truth):
    n, c, h, w = feat_t0.shape
    npix = h * w
    f0 = feat_t0.reshape(c, npix)
    f1 = feat_t1.reshape(c, npix)
    gt = ground_truth.reshape(1, npix)
    grid = npix // _CHUNK
    out = pl.pallas_call(
        _hist_loss_kernel,
        grid=(grid,),
        in_specs=[
            pl.BlockSpec((c, _CHUNK), lambda i: (0, i)),
            pl.BlockSpec((c, _CHUNK), lambda i: (0, i)),
            pl.BlockSpec((1, _CHUNK), lambda i: (0, i)),
        ],
        out_specs=pl.BlockSpec((1, 1), lambda i: (0, 0)),
        out_shape=jax.ShapeDtypeStruct((1, 1), jnp.float32),
        scratch_shapes=[
            pltpu.VMEM((_BINS, 1), jnp.float32),
            pltpu.VMEM((_BINS, 1), jnp.float32),
            pltpu.VMEM((1, 1), jnp.float32),
        ],
        compiler_params=pltpu.CompilerParams(
            dimension_semantics=("arbitrary",),
        ),
    )(f0, f1, gt)
    return out[0, 0]


# speedup vs baseline: 5.8464x; 2.7783x over previous
import jax
import jax.numpy as jnp
from jax.experimental import pallas as pl
from jax.experimental.pallas import tpu as pltpu

_BINS = 100
_ROWS = 16


def _hist_loss_kernel(f0_ref, f1_ref, gt_ref, out_ref, hp_ref, ha_ref, sz_ref):
    i = pl.program_id(0)
    nsteps = pl.num_programs(0)

    @pl.when(i == 0)
    def _init():
        hp_ref[...] = jnp.zeros_like(hp_ref)
        ha_ref[...] = jnp.zeros_like(ha_ref)
        sz_ref[...] = jnp.zeros_like(sz_ref)

    t = f0_ref[...] + 1e-6 - f1_ref[...]
    dist2 = jnp.sum(t * t, axis=0)
    dist = jnp.sqrt(dist2)
    gt = gt_ref[...]
    pos = (gt == 0).astype(jnp.float32)
    in_range = (dist <= 1.0).astype(jnp.float32)
    idx = jnp.clip(jnp.floor(dist * float(_BINS)).astype(jnp.int32), 0, _BINS - 1)
    bins = jax.lax.broadcasted_iota(jnp.int32, (_BINS, 1, 1), 0)
    onehot = idx[None, :, :] == bins
    w_pos = (pos * in_range)[None, :, :]
    w_all = in_range[None, :, :]
    hp_ref[...] += jnp.sum(jnp.where(onehot, w_pos, 0.0), axis=1)
    ha_ref[...] += jnp.sum(jnp.where(onehot, w_all, 0.0), axis=1)
    sz_ref[...] += jnp.sum(pos).reshape(1, 1)

    @pl.when(i == nsteps - 1)
    def _finalize():
        npix = nsteps * _ROWS * gt_ref.shape[-1]
        pos_size = sz_ref[0, 0]
        neg_size = float(npix) - pos_size
        hps = jnp.sum(hp_ref[...], axis=1, keepdims=True)
        ha = jnp.sum(ha_ref[...], axis=1, keepdims=True)
        hp = hps / pos_size
        hn = (ha - hps) / neg_size
        pointwise = jnp.where(hn > 0, hn * (jnp.log(hn) - hp), 0.0)
        out_ref[...] = (jnp.sum(pointwise) / float(_BINS) + 1.0).reshape(1, 1)


@jax.jit
def kernel(feat_t0, feat_t1, ground_truth):
    n, c, h, w = feat_t0.shape
    f0 = feat_t0.reshape(c, h, w)
    f1 = feat_t1.reshape(c, h, w)
    grid = h // _ROWS
    out = pl.pallas_call(
        _hist_loss_kernel,
        grid=(grid,),
        in_specs=[
            pl.BlockSpec((c, _ROWS, w), lambda i: (0, i, 0)),
            pl.BlockSpec((c, _ROWS, w), lambda i: (0, i, 0)),
            pl.BlockSpec((_ROWS, w), lambda i: (i, 0)),
        ],
        out_specs=pl.BlockSpec((1, 1), lambda i: (0, 0)),
        out_shape=jax.ShapeDtypeStruct((1, 1), jnp.float32),
        scratch_shapes=[
            pltpu.VMEM((_BINS, w), jnp.float32),
            pltpu.VMEM((_BINS, w), jnp.float32),
            pltpu.VMEM((1, 1), jnp.float32),
        ],
        compiler_params=pltpu.CompilerParams(
            dimension_semantics=("arbitrary",),
        ),
    )(f0, f1, ground_truth)
    return out[0, 0]


# speedup vs baseline: 6.6170x; 1.1318x over previous
import jax
import jax.numpy as jnp
from jax.experimental import pallas as pl
from jax.experimental.pallas import tpu as pltpu

_BINS = 100
_ROWS = 32


def _hist_loss_kernel(f0_ref, f1_ref, gt_ref, out_ref, hp_ref, ha_ref, sz_ref):
    i = pl.program_id(0)
    nsteps = pl.num_programs(0)

    @pl.when(i == 0)
    def _init():
        hp_ref[...] = jnp.zeros_like(hp_ref)
        ha_ref[...] = jnp.zeros_like(ha_ref)
        sz_ref[...] = jnp.zeros_like(sz_ref)

    t = f0_ref[...] + 1e-6 - f1_ref[...]
    dist2 = jnp.sum(t * t, axis=0)
    dist = jnp.sqrt(dist2)
    gt = gt_ref[...]
    pos = (gt == 0).astype(jnp.float32)
    in_range = (dist <= 1.0).astype(jnp.float32)
    idx = jnp.clip(jnp.floor(dist * float(_BINS)).astype(jnp.int32), 0, _BINS - 1)
    bins = jax.lax.broadcasted_iota(jnp.int32, (_BINS, 1, 1), 0)
    onehot = idx[None, :, :] == bins
    w_pos = (pos * in_range)[None, :, :]
    w_all = in_range[None, :, :]
    hp_ref[...] += jnp.sum(jnp.where(onehot, w_pos, 0.0), axis=1)
    ha_ref[...] += jnp.sum(jnp.where(onehot, w_all, 0.0), axis=1)
    sz_ref[...] += jnp.sum(pos).reshape(1, 1)

    @pl.when(i == nsteps - 1)
    def _finalize():
        npix = nsteps * _ROWS * gt_ref.shape[-1]
        pos_size = sz_ref[0, 0]
        neg_size = float(npix) - pos_size
        hps = jnp.sum(hp_ref[...], axis=1, keepdims=True)
        ha = jnp.sum(ha_ref[...], axis=1, keepdims=True)
        hp = hps / pos_size
        hn = (ha - hps) / neg_size
        pointwise = jnp.where(hn > 0, hn * (jnp.log(hn) - hp), 0.0)
        out_ref[...] = (jnp.sum(pointwise) / float(_BINS) + 1.0).reshape(1, 1)


@jax.jit
def kernel(feat_t0, feat_t1, ground_truth):
    n, c, h, w = feat_t0.shape
    f0 = feat_t0.reshape(c, h, w)
    f1 = feat_t1.reshape(c, h, w)
    grid = h // _ROWS
    out = pl.pallas_call(
        _hist_loss_kernel,
        grid=(grid,),
        in_specs=[
            pl.BlockSpec((c, _ROWS, w), lambda i: (0, i, 0)),
            pl.BlockSpec((c, _ROWS, w), lambda i: (0, i, 0)),
            pl.BlockSpec((_ROWS, w), lambda i: (i, 0)),
        ],
        out_specs=pl.BlockSpec((1, 1), lambda i: (0, 0)),
        out_shape=jax.ShapeDtypeStruct((1, 1), jnp.float32),
        scratch_shapes=[
            pltpu.VMEM((_BINS, w), jnp.float32),
            pltpu.VMEM((_BINS, w), jnp.float32),
            pltpu.VMEM((1, 1), jnp.float32),
        ],
        compiler_params=pltpu.CompilerParams(
            dimension_semantics=("arbitrary",),
        ),
    )(f0, f1, ground_truth)
    return out[0, 0]


# speedup vs baseline: 6.8679x; 1.0379x over previous
import jax
import jax.numpy as jnp
from jax.experimental import pallas as pl
from jax.experimental.pallas import tpu as pltpu

_BINS = 100
_ROWS = 32


def _hist_loss_kernel(f0_ref, f1_ref, gt_ref, out_ref, hp_ref, ha_ref, sz_ref):
    i = pl.program_id(0)
    nsteps = pl.num_programs(0)

    @pl.when(i == 0)
    def _init():
        hp_ref[...] = jnp.zeros_like(hp_ref)
        ha_ref[...] = jnp.zeros_like(ha_ref)
        sz_ref[...] = jnp.zeros_like(sz_ref)

    c = f0_ref.shape[0]
    r, w = gt_ref.shape
    dist2 = jnp.zeros((r, w), jnp.float32)
    for k0 in range(0, c, 8):
        t = f0_ref[k0:k0 + 8] + 1e-6 - f1_ref[k0:k0 + 8]
        dist2 = dist2 + jnp.sum(t * t, axis=0)
    dist = jnp.sqrt(dist2)
    gt = gt_ref[...]
    pos = (gt == 0).astype(jnp.float32)
    raw = jnp.minimum(jnp.floor(dist * float(_BINS)), float(_BINS)).astype(jnp.int32)
    idx = jnp.where(dist <= 1.0, jnp.minimum(raw, _BINS - 1), _BINS)
    bins = jax.lax.broadcasted_iota(jnp.int32, (_BINS + 1, 1, 1), 0)
    onehot = idx[None, :, :] == bins
    hp_ref[...] += jnp.sum(jnp.where(onehot, pos[None, :, :], 0.0), axis=1)
    ha_ref[...] += jnp.sum(onehot.astype(jnp.float32), axis=1)
    sz_ref[...] += jnp.sum(pos).reshape(1, 1)

    @pl.when(i == nsteps - 1)
    def _finalize():
        npix = nsteps * _ROWS * gt_ref.shape[-1]
        pos_size = sz_ref[0, 0]
        neg_size = float(npix) - pos_size
        hps = jnp.sum(hp_ref[:_BINS], axis=1, keepdims=True)
        ha = jnp.sum(ha_ref[:_BINS], axis=1, keepdims=True)
        hp = hps / pos_size
        hn = (ha - hps) / neg_size
        pointwise = jnp.where(hn > 0, hn * (jnp.log(hn) - hp), 0.0)
        out_ref[...] = (jnp.sum(pointwise) / float(_BINS) + 1.0).reshape(1, 1)


@jax.jit
def kernel(feat_t0, feat_t1, ground_truth):
    n, c, h, w = feat_t0.shape
    f0 = feat_t0.reshape(c, h, w)
    f1 = feat_t1.reshape(c, h, w)
    grid = h // _ROWS
    out = pl.pallas_call(
        _hist_loss_kernel,
        grid=(grid,),
        in_specs=[
            pl.BlockSpec((c, _ROWS, w), lambda i: (0, i, 0)),
            pl.BlockSpec((c, _ROWS, w), lambda i: (0, i, 0)),
            pl.BlockSpec((_ROWS, w), lambda i: (i, 0)),
        ],
        out_specs=pl.BlockSpec((1, 1), lambda i: (0, 0)),
        out_shape=jax.ShapeDtypeStruct((1, 1), jnp.float32),
        scratch_shapes=[
            pltpu.VMEM((_BINS + 1, w), jnp.float32),
            pltpu.VMEM((_BINS + 1, w), jnp.float32),
            pltpu.VMEM((1, 1), jnp.float32),
        ],
        compiler_params=pltpu.CompilerParams(
            dimension_semantics=("arbitrary",),
        ),
    )(f0, f1, ground_truth)
    return out[0, 0]


# speedup vs baseline: 6.9182x; 1.0073x over previous
import jax
import jax.numpy as jnp
from jax.experimental import pallas as pl
from jax.experimental.pallas import tpu as pltpu

_BINS = 100
_ROWS = 32


def _hist_loss_kernel(f0_ref, f1_ref, gt_ref, out_ref, h_ref):
    i = pl.program_id(0)
    nsteps = pl.num_programs(0)

    @pl.when(i == 0)
    def _init():
        h_ref[...] = jnp.zeros_like(h_ref)

    c = f0_ref.shape[0]
    r, w = gt_ref.shape
    dist2 = jnp.zeros((r, w), jnp.float32)
    for k0 in range(0, c, 8):
        t = f0_ref[k0:k0 + 8] + 1e-6 - f1_ref[k0:k0 + 8]
        dist2 = dist2 + jnp.sum(t * t, axis=0)
    dist = jnp.sqrt(dist2)
    gt = gt_ref[...]
    raw = jnp.minimum(jnp.floor(dist * float(_BINS)), float(_BINS)).astype(jnp.int32)
    idx = jnp.where(dist <= 1.0, jnp.minimum(raw, _BINS - 1), _BINS)
    idx = jnp.where(gt == 0, idx, idx + (_BINS + 1))
    bins = jax.lax.broadcasted_iota(jnp.int32, (2 * _BINS + 2, 1, 1), 0)
    onehot = idx[None, :, :] == bins
    h_ref[...] += jnp.sum(onehot.astype(jnp.float32), axis=1)

    @pl.when(i == nsteps - 1)
    def _finalize():
        npix = nsteps * _ROWS * gt_ref.shape[-1]
        pos_size = jnp.sum(h_ref[0:_BINS + 1])
        neg_size = float(npix) - pos_size
        hps = jnp.sum(h_ref[0:_BINS], axis=1, keepdims=True)
        hns = jnp.sum(h_ref[_BINS + 1:2 * _BINS + 1], axis=1, keepdims=True)
        hp = hps / pos_size
        hn = hns / neg_size
        pointwise = jnp.where(hn > 0, hn * (jnp.log(hn) - hp), 0.0)
        out_ref[...] = (jnp.sum(pointwise) / float(_BINS) + 1.0).reshape(1, 1)


@jax.jit
def kernel(feat_t0, feat_t1, ground_truth):
    n, c, h, w = feat_t0.shape
    f0 = feat_t0.reshape(c, h, w)
    f1 = feat_t1.reshape(c, h, w)
    grid = h // _ROWS
    out = pl.pallas_call(
        _hist_loss_kernel,
        grid=(grid,),
        in_specs=[
            pl.BlockSpec((c, _ROWS, w), lambda i: (0, i, 0)),
            pl.BlockSpec((c, _ROWS, w), lambda i: (0, i, 0)),
            pl.BlockSpec((_ROWS, w), lambda i: (i, 0)),
        ],
        out_specs=pl.BlockSpec((1, 1), lambda i: (0, 0)),
        out_shape=jax.ShapeDtypeStruct((1, 1), jnp.float32),
        scratch_shapes=[
            pltpu.VMEM((2 * _BINS + 2, w), jnp.float32),
        ],
        compiler_params=pltpu.CompilerParams(
            dimension_semantics=("arbitrary",),
        ),
    )(f0, f1, ground_truth)
    return out[0, 0]


# speedup vs baseline: 7.6708x; 1.1088x over previous
import jax
import jax.numpy as jnp
from jax.experimental import pallas as pl
from jax.experimental.pallas import tpu as pltpu

_BINS = 100
_ROWS = 32


def _hist_loss_kernel(f0_ref, f1_ref, gt_ref, out_ref, h_ref):
    i = pl.program_id(0)
    nsteps = pl.num_programs(0)

    @pl.when(i == 0)
    def _init():
        h_ref[...] = jnp.zeros_like(h_ref)

    c = f0_ref.shape[0]
    r, w = gt_ref.shape
    dist2 = jnp.zeros((r, w), jnp.float32)
    for k0 in range(0, c, 8):
        t = f0_ref[k0:k0 + 8] + 1e-6 - f1_ref[k0:k0 + 8]
        dist2 = dist2 + jnp.sum(t * t, axis=0)
    dist = jnp.sqrt(dist2)
    gt = gt_ref[...]
    raw = jnp.minimum(jnp.floor(dist * float(_BINS)), float(_BINS)).astype(jnp.int32)
    idx = jnp.where(dist <= 1.0, jnp.minimum(raw, _BINS - 1), _BINS)
    val = jnp.where(gt == 0, 1, 1 << 16)
    bins = jax.lax.broadcasted_iota(jnp.int32, (_BINS + 1, 1, 1), 0)
    onehot = idx[None, :, :] == bins
    h_ref[...] += jnp.sum(jnp.where(onehot, val[None, :, :], 0), axis=1)

    @pl.when(i == nsteps - 1)
    def _finalize():
        npix = nsteps * _ROWS * gt_ref.shape[-1]
        hcells = h_ref[...]
        pcells = (hcells & 0xFFFF).astype(jnp.float32)
        ncells = (hcells >> 16).astype(jnp.float32)
        pos_size = jnp.sum(pcells)
        neg_size = float(npix) - pos_size
        hps = jnp.sum(pcells[0:_BINS], axis=1, keepdims=True)
        hns = jnp.sum(ncells[0:_BINS], axis=1, keepdims=True)
        hp = hps / pos_size
        hn = hns / neg_size
        pointwise = jnp.where(hn > 0, hn * (jnp.log(hn) - hp), 0.0)
        out_ref[...] = (jnp.sum(pointwise) / float(_BINS) + 1.0).reshape(1, 1)


@jax.jit
def kernel(feat_t0, feat_t1, ground_truth):
    n, c, h, w = feat_t0.shape
    f0 = feat_t0.reshape(c, h, w)
    f1 = feat_t1.reshape(c, h, w)
    grid = h // _ROWS
    out = pl.pallas_call(
        _hist_loss_kernel,
        grid=(grid,),
        in_specs=[
            pl.BlockSpec((c, _ROWS, w), lambda i: (0, i, 0)),
            pl.BlockSpec((c, _ROWS, w), lambda i: (0, i, 0)),
            pl.BlockSpec((_ROWS, w), lambda i: (i, 0)),
        ],
        out_specs=pl.BlockSpec((1, 1), lambda i: (0, 0)),
        out_shape=jax.ShapeDtypeStruct((1, 1), jnp.float32),
        scratch_shapes=[
            pltpu.VMEM((_BINS + 1, w), jnp.int32),
        ],
        compiler_params=pltpu.CompilerParams(
            dimension_semantics=("arbitrary",),
        ),
    )(f0, f1, ground_truth)
    return out[0, 0]
